# Initial kernel scaffold; baseline (speedup 1.0000x reference)
import jax
import jax.numpy as jnp
from jax.experimental import pallas as pl


def kernel(x, edge_index, edge_weight, W1, b1, g1, be1, W2, b2, g2, be2, W3, b3):
    raise NotImplementedError("write your pallas kernel here")


# speedup vs baseline: 7.0399x; 7.0399x over previous
import functools

import jax
import jax.numpy as jnp
from jax import lax
from jax.experimental import pallas as pl
from jax.experimental.pallas import tpu as pltpu
from jax.experimental.pallas import tpu_sc as plsc

N = 10000
E = 320000
D = 128

NC = 2
NS = 16
NW = NC * NS
EPT = 10240
EPAD = NW * EPT
CH = 128
NCHUNK = EPT // CH
NPAD = 10240
RPT = NPAD // NS

_mesh = plsc.VectorSubcoreMesh(core_axis_name="c", subcore_axis_name="s")
_f32 = jnp.float32


def _deg_body(dst3, w3, out, degacc, dstv, wv, zvec):
    c = lax.axis_index("c")
    s = lax.axis_index("s")
    wid = c * NS + s

    def zrow(i, carry):
        zvec[pl.ds(i * 16, 16)] = jnp.zeros((16,), _f32)
        return carry

    lax.fori_loop(0, RPT // 16, zrow, 0)
    pltpu.sync_copy(zvec, degacc.at[pl.ds(s * RPT, RPT)])
    plsc.subcore_barrier()

    pltpu.sync_copy(dst3.at[wid], dstv)
    pltpu.sync_copy(w3.at[wid], wv)

    def chunk(k, carry):
        pltpu.sync_copy(wv.at[k], degacc.at[dstv.at[k]], add=True)
        return carry

    lax.fori_loop(0, NCHUNK, chunk, 0)
    plsc.subcore_barrier()
    pltpu.sync_copy(degacc.at[pl.ds(s * RPT, RPT)], out.at[c, pl.ds(s * RPT, RPT)])


_deg_call = functools.partial(
    pl.kernel,
    out_type=jax.ShapeDtypeStruct((NC, NPAD), _f32),
    mesh=_mesh,
    scratch_types=[
        pltpu.VMEM_SHARED((NPAD,), _f32),
        pltpu.VMEM((NCHUNK, CH), jnp.int32),
        pltpu.VMEM((NCHUNK, CH), _f32),
        pltpu.VMEM((RPT,), _f32),
    ],
)(_deg_body)


GRP = 8
NGRP = NCHUNK // GRP


def _agg_body(hs, src4, dst4, w4, out, acc, srcv, dstv, wv, rows, gsem):
    c = lax.axis_index("c")
    s = lax.axis_index("s")
    wid = c * NS + s

    z16 = jnp.zeros((16,), _f32)

    def zrow(i, carry):
        for f in range(D // 16):
            rows[i, pl.ds(f * 16, 16)] = z16
        return carry

    lax.fori_loop(0, CH, zrow, 0)
    for j in range(RPT // CH):
        pltpu.sync_copy(rows, acc.at[pl.ds(s * RPT + j * CH, CH)])
    plsc.subcore_barrier()

    def grp(q, carry):
        pltpu.sync_copy(src4.at[wid, q], srcv)
        pltpu.sync_copy(dst4.at[wid, q], dstv)
        pltpu.sync_copy(w4.at[wid, q], wv)

        def chunk(k, c1):
            pltpu.async_copy(hs.at[srcv.at[k]], rows, gsem).wait()

            def group(g, c2):
                wvec = wv[k, pl.ds(g * 16, 16)]
                base = g * 16
                for e in range(16):
                    sc = wvec[e]
                    for f in range(D // 16):
                        rows[base + e, pl.ds(f * 16, 16)] = (
                            rows[base + e, pl.ds(f * 16, 16)] * sc
                        )
                return c2

            lax.fori_loop(0, CH // 16, group, 0)
            pltpu.sync_copy(rows, acc.at[dstv.at[k]], add=True)
            return c1

        lax.fori_loop(0, GRP, chunk, 0)
        return carry

    lax.fori_loop(0, NGRP, grp, 0)
    plsc.subcore_barrier()
    pltpu.sync_copy(acc.at[pl.ds(s * RPT, RPT)], out.at[c, pl.ds(s * RPT, RPT)])


_agg_call = functools.partial(
    pl.kernel,
    out_type=jax.ShapeDtypeStruct((NC, NPAD, D), _f32),
    mesh=_mesh,
    scratch_types=[
        pltpu.VMEM_SHARED((NPAD, D), _f32),
        pltpu.VMEM((GRP, CH), jnp.int32),
        pltpu.VMEM((GRP, CH), jnp.int32),
        pltpu.VMEM((GRP, CH), _f32),
        pltpu.VMEM((CH, D), _f32),
        pltpu.SemaphoreType.DMA,
    ],
)(_agg_body)


def _k0_body(x_ref, w_ref, degp_ref, hs_ref, dinv_ref):
    deg = jnp.sum(degp_ref[...], axis=0) + 1.0
    dinv = jnp.where(deg > 0, lax.rsqrt(deg), 0.0)
    h = jnp.dot(x_ref[...], w_ref[...], preferred_element_type=_f32)
    hs_ref[...] = h * dinv[:N, :]
    dinv_ref[...] = dinv


def _k0_call(x, w1, degp3):
    return pl.pallas_call(
        _k0_body,
        out_shape=(
            jax.ShapeDtypeStruct((N, D), _f32),
            jax.ShapeDtypeStruct((NPAD, 1), _f32),
        ),
    )(x, w1, degp3)


def _trans_body(parts_ref, hs_ref, dinv_ref, b_ref, g_ref, be_ref, w_ref, out_ref):
    dv = dinv_ref[:N, :]
    t = dv * (parts_ref[0, :N, :] + parts_ref[1, :N, :] + hs_ref[...]) + b_ref[...]
    r = jnp.maximum(t, 0.0)
    mu = jnp.mean(r, axis=0, keepdims=True)
    var = jnp.mean((r - mu) ** 2, axis=0, keepdims=True)
    y = (r - mu) * lax.rsqrt(var + 1e-5) * g_ref[...] + be_ref[...]
    h = jnp.dot(y, w_ref[...], preferred_element_type=_f32)
    out_ref[...] = h * dv


def _trans_call(parts, hs, dinv, b, g, be, w):
    return pl.pallas_call(
        _trans_body,
        out_shape=jax.ShapeDtypeStruct((N, D), _f32),
    )(parts, hs, dinv, b, g, be, w)


def _final_body(parts_ref, hs_ref, dinv_ref, b_ref, out_ref):
    dv = dinv_ref[:N, :]
    out_ref[...] = (
        dv * (parts_ref[0, :N, :] + parts_ref[1, :N, :] + hs_ref[...]) + b_ref[...]
    )


def _final_call(parts, hs, dinv, b):
    return pl.pallas_call(
        _final_body,
        out_shape=jax.ShapeDtypeStruct((N, D), _f32),
    )(parts, hs, dinv, b)


def kernel(x, edge_index, edge_weight, W1, b1, g1, be1, W2, b2, g2, be2, W3, b3):
    src = edge_index[0]
    dst = edge_index[1]
    pad = EPAD - E
    zi = jnp.zeros((pad,), jnp.int32)
    srcp = jnp.concatenate([src, zi])
    dstp = jnp.concatenate([dst, zi])
    wp = jnp.concatenate([edge_weight, jnp.zeros((pad,), _f32)])
    dst3 = dstp.reshape(NW, NCHUNK, CH)
    w3 = wp.reshape(NW, NCHUNK, CH)
    src4 = srcp.reshape(NW, NGRP, GRP, CH)
    dst4 = dstp.reshape(NW, NGRP, GRP, CH)
    w4 = wp.reshape(NW, NGRP, GRP, CH)
    b1r, g1r, be1r = b1.reshape(1, D), g1.reshape(1, D), be1.reshape(1, D)
    b2r, g2r, be2r = b2.reshape(1, D), g2.reshape(1, D), be2.reshape(1, D)
    b3r = b3.reshape(1, D)

    deg_parts = _deg_call(dst3, w3)
    degp3 = deg_parts.reshape(NC, NPAD, 1)
    h1s, dinv = _k0_call(x, W1, degp3)
    parts = _agg_call(h1s, src4, dst4, w4)
    h2s = _trans_call(parts, h1s, dinv, b1r, g1r, be1r, W2)
    parts = _agg_call(h2s, src4, dst4, w4)
    h3s = _trans_call(parts, h2s, dinv, b2r, g2r, be2r, W3)
    parts = _agg_call(h3s, src4, dst4, w4)
    return _final_call(parts, h3s, dinv, b3r)


# speedup vs baseline: 8.4649x; 1.2024x over previous
import functools

import jax
import jax.numpy as jnp
from jax import lax
from jax.experimental import pallas as pl
from jax.experimental.pallas import tpu as pltpu
from jax.experimental.pallas import tpu_sc as plsc

N = 10000
E = 320000
D = 128

NC = 2
NS = 16
NW = NC * NS
EPT = 10240
EPAD = NW * EPT
CH = 128
NCHUNK = EPT // CH
NPAD = 10240
RPT = NPAD // NS

_mesh = plsc.VectorSubcoreMesh(core_axis_name="c", subcore_axis_name="s")
_f32 = jnp.float32


def _deg_body(dst3, w3, out, degacc, dstv, wv, zvec):
    c = lax.axis_index("c")
    s = lax.axis_index("s")
    wid = c * NS + s

    def zrow(i, carry):
        zvec[pl.ds(i * 16, 16)] = jnp.zeros((16,), _f32)
        return carry

    lax.fori_loop(0, RPT // 16, zrow, 0)
    pltpu.sync_copy(zvec, degacc.at[pl.ds(s * RPT, RPT)])
    plsc.subcore_barrier()

    pltpu.sync_copy(dst3.at[wid], dstv)
    pltpu.sync_copy(w3.at[wid], wv)

    def chunk(k, carry):
        pltpu.sync_copy(wv.at[k], degacc.at[dstv.at[k]], add=True)
        return carry

    lax.fori_loop(0, NCHUNK, chunk, 0)
    plsc.subcore_barrier()
    pltpu.sync_copy(degacc.at[pl.ds(s * RPT, RPT)], out.at[c, pl.ds(s * RPT, RPT)])


_deg_call = functools.partial(
    pl.kernel,
    out_type=jax.ShapeDtypeStruct((NC, NPAD), _f32),
    mesh=_mesh,
    scratch_types=[
        pltpu.VMEM_SHARED((NPAD,), _f32),
        pltpu.VMEM((NCHUNK, CH), jnp.int32),
        pltpu.VMEM((NCHUNK, CH), _f32),
        pltpu.VMEM((RPT,), _f32),
    ],
)(_deg_body)


GRP = 16
NGRP = NCHUNK // GRP
NPAIR = GRP // 2


def _agg_body(hs, src4, dst4, w4, out, acc,
              srcm0, dstm0, wm0, srcm1, dstm1, wm1,
              rowsA, rowsB, gsA, gsB, ssA, ssB, ms0, ms1):
    c = lax.axis_index("c")
    s = lax.axis_index("s")
    wid = c * NS + s
    metas = ((srcm0, dstm0, wm0, ms0), (srcm1, dstm1, wm1, ms1))

    z16 = jnp.zeros((16,), _f32)

    def zrow(i, carry):
        for f in range(D // 16):
            rowsA[i, pl.ds(f * 16, 16)] = z16
        return carry

    lax.fori_loop(0, CH, zrow, 0)
    for j in range(RPT // CH):
        pltpu.sync_copy(rowsA, acc.at[pl.ds(s * RPT + j * CH, CH)])
    plsc.subcore_barrier()

    def fetch_meta(q, p):
        sm, dm, wm, msem = metas[p]
        pltpu.async_copy(src4.at[wid, q], sm, msem)
        pltpu.async_copy(dst4.at[wid, q], dm, msem)
        pltpu.async_copy(w4.at[wid, q], wm, msem)

    def wait_meta(q, p):
        sm, dm, wm, msem = metas[p]
        pltpu.make_async_copy(src4.at[wid, q], sm, msem).wait()
        pltpu.make_async_copy(dst4.at[wid, q], dm, msem).wait()
        pltpu.make_async_copy(w4.at[wid, q], wm, msem).wait()

    def scale(rows, wm, k):
        def group(g, c2):
            wvec = wm[k, pl.ds(g * 16, 16)]
            base = g * 16
            for e in range(16):
                sc = wvec[e]
                for f in range(D // 16):
                    rows[base + e, pl.ds(f * 16, 16)] = (
                        rows[base + e, pl.ds(f * 16, 16)] * sc
                    )
            return c2

        lax.fori_loop(0, CH // 16, group, 0)

    fetch_meta(0, 0)
    for q in range(NGRP):
        p = q & 1
        sm, dm, wm, msem = metas[p]
        wait_meta(q, p)
        pltpu.async_copy(hs.at[sm.at[0]], rowsA, gsA)
        if q + 1 < NGRP:
            fetch_meta(q + 1, 1 - p)

        def pair(j, c1):
            k0 = 2 * j
            k1 = 2 * j + 1

            @pl.when(j > 0)
            def _():
                pltpu.make_async_copy(rowsB, acc.at[dm.at[k1]], ssB).wait()

            pltpu.make_async_copy(hs.at[sm.at[k0]], rowsA, gsA).wait()
            pltpu.async_copy(hs.at[sm.at[k1]], rowsB, gsB)
            scale(rowsA, wm, k0)
            pltpu.async_copy(rowsA, acc.at[dm.at[k0]], ssA, add=True)
            pltpu.make_async_copy(hs.at[sm.at[k1]], rowsB, gsB).wait()
            pltpu.make_async_copy(rowsA, acc.at[dm.at[k0]], ssA).wait()

            @pl.when(j < NPAIR - 1)
            def _():
                pltpu.async_copy(hs.at[sm.at[k0 + 2]], rowsA, gsA)

            scale(rowsB, wm, k1)
            pltpu.async_copy(rowsB, acc.at[dm.at[k1]], ssB, add=True)
            return c1

        lax.fori_loop(0, NPAIR, pair, 0)
        pltpu.make_async_copy(rowsB, acc.at[dm.at[0]], ssB).wait()

    plsc.subcore_barrier()
    pltpu.sync_copy(acc.at[pl.ds(s * RPT, RPT)], out.at[c, pl.ds(s * RPT, RPT)])


_agg_call = functools.partial(
    pl.kernel,
    out_type=jax.ShapeDtypeStruct((NC, NPAD, D), _f32),
    mesh=_mesh,
    scratch_types=[
        pltpu.VMEM_SHARED((NPAD, D), _f32),
        pltpu.VMEM((GRP, CH), jnp.int32),
        pltpu.VMEM((GRP, CH), jnp.int32),
        pltpu.VMEM((GRP, CH), _f32),
        pltpu.VMEM((GRP, CH), jnp.int32),
        pltpu.VMEM((GRP, CH), jnp.int32),
        pltpu.VMEM((GRP, CH), _f32),
        pltpu.VMEM((CH, D), _f32),
        pltpu.VMEM((CH, D), _f32),
        pltpu.SemaphoreType.DMA,
        pltpu.SemaphoreType.DMA,
        pltpu.SemaphoreType.DMA,
        pltpu.SemaphoreType.DMA,
        pltpu.SemaphoreType.DMA,
        pltpu.SemaphoreType.DMA,
    ],
)(_agg_body)


def _k0_body(x_ref, w_ref, degp_ref, hs_ref, dinv_ref):
    deg = jnp.sum(degp_ref[...], axis=0) + 1.0
    dinv = jnp.where(deg > 0, lax.rsqrt(deg), 0.0)
    h = jnp.dot(x_ref[...], w_ref[...], preferred_element_type=_f32)
    hs_ref[...] = h * dinv[:N, :]
    dinv_ref[...] = dinv


def _k0_call(x, w1, degp3):
    return pl.pallas_call(
        _k0_body,
        out_shape=(
            jax.ShapeDtypeStruct((N, D), _f32),
            jax.ShapeDtypeStruct((NPAD, 1), _f32),
        ),
    )(x, w1, degp3)


def _trans_body(parts_ref, hs_ref, dinv_ref, b_ref, g_ref, be_ref, w_ref, out_ref):
    dv = dinv_ref[:N, :]
    t = dv * (parts_ref[0, :N, :] + parts_ref[1, :N, :] + hs_ref[...]) + b_ref[...]
    r = jnp.maximum(t, 0.0)
    mu = jnp.mean(r, axis=0, keepdims=True)
    var = jnp.mean((r - mu) ** 2, axis=0, keepdims=True)
    y = (r - mu) * lax.rsqrt(var + 1e-5) * g_ref[...] + be_ref[...]
    h = jnp.dot(y, w_ref[...], preferred_element_type=_f32)
    out_ref[...] = h * dv


def _trans_call(parts, hs, dinv, b, g, be, w):
    return pl.pallas_call(
        _trans_body,
        out_shape=jax.ShapeDtypeStruct((N, D), _f32),
    )(parts, hs, dinv, b, g, be, w)


def _final_body(parts_ref, hs_ref, dinv_ref, b_ref, out_ref):
    dv = dinv_ref[:N, :]
    out_ref[...] = (
        dv * (parts_ref[0, :N, :] + parts_ref[1, :N, :] + hs_ref[...]) + b_ref[...]
    )


def _final_call(parts, hs, dinv, b):
    return pl.pallas_call(
        _final_body,
        out_shape=jax.ShapeDtypeStruct((N, D), _f32),
    )(parts, hs, dinv, b)


def kernel(x, edge_index, edge_weight, W1, b1, g1, be1, W2, b2, g2, be2, W3, b3):
    src = edge_index[0]
    dst = edge_index[1]
    pad = EPAD - E
    zi = jnp.zeros((pad,), jnp.int32)
    srcp = jnp.concatenate([src, zi])
    dstp = jnp.concatenate([dst, zi])
    wp = jnp.concatenate([edge_weight, jnp.zeros((pad,), _f32)])
    dst3 = dstp.reshape(NW, NCHUNK, CH)
    w3 = wp.reshape(NW, NCHUNK, CH)
    src4 = srcp.reshape(NW, NGRP, GRP, CH)
    dst4 = dstp.reshape(NW, NGRP, GRP, CH)
    w4 = wp.reshape(NW, NGRP, GRP, CH)
    b1r, g1r, be1r = b1.reshape(1, D), g1.reshape(1, D), be1.reshape(1, D)
    b2r, g2r, be2r = b2.reshape(1, D), g2.reshape(1, D), be2.reshape(1, D)
    b3r = b3.reshape(1, D)

    deg_parts = _deg_call(dst3, w3)
    degp3 = deg_parts.reshape(NC, NPAD, 1)
    h1s, dinv = _k0_call(x, W1, degp3)
    parts = _agg_call(h1s, src4, dst4, w4)
    h2s = _trans_call(parts, h1s, dinv, b1r, g1r, be1r, W2)
    parts = _agg_call(h2s, src4, dst4, w4)
    h3s = _trans_call(parts, h2s, dinv, b2r, g2r, be2r, W3)
    parts = _agg_call(h3s, src4, dst4, w4)
    return _final_call(parts, h3s, dinv, b3r)


# speedup vs baseline: 8.5278x; 1.0074x over previous
import functools

import jax
import jax.numpy as jnp
from jax import lax
from jax.experimental import pallas as pl
from jax.experimental.pallas import tpu as pltpu
from jax.experimental.pallas import tpu_sc as plsc

N = 10000
E = 320000
D = 128

NC = 2
NS = 16
NW = NC * NS
EPT = 10240
EPAD = NW * EPT
CH = 128
NCHUNK = EPT // CH
NPAD = 10240
RPT = NPAD // NS

_mesh = plsc.VectorSubcoreMesh(core_axis_name="c", subcore_axis_name="s")
_f32 = jnp.float32


def _deg_body(dst3, w3, out, degacc, dstv, wv, zvec):
    c = lax.axis_index("c")
    s = lax.axis_index("s")
    wid = c * NS + s

    def zrow(i, carry):
        zvec[pl.ds(i * 16, 16)] = jnp.zeros((16,), _f32)
        return carry

    lax.fori_loop(0, RPT // 16, zrow, 0)
    pltpu.sync_copy(zvec, degacc.at[pl.ds(s * RPT, RPT)])
    plsc.subcore_barrier()

    pltpu.sync_copy(dst3.at[wid], dstv)
    pltpu.sync_copy(w3.at[wid], wv)

    def chunk(k, carry):
        pltpu.sync_copy(wv.at[k], degacc.at[dstv.at[k]], add=True)
        return carry

    lax.fori_loop(0, NCHUNK, chunk, 0)
    plsc.subcore_barrier()
    pltpu.sync_copy(degacc.at[pl.ds(s * RPT, RPT)], out.at[c, pl.ds(s * RPT, RPT)])


_deg_call = functools.partial(
    pl.kernel,
    out_type=jax.ShapeDtypeStruct((NC, NPAD), _f32),
    mesh=_mesh,
    scratch_types=[
        pltpu.VMEM_SHARED((NPAD,), _f32),
        pltpu.VMEM((NCHUNK, CH), jnp.int32),
        pltpu.VMEM((NCHUNK, CH), _f32),
        pltpu.VMEM((RPT,), _f32),
    ],
)(_deg_body)


GRP = 16
NPAIR = GRP // 2
NG_SLOW = 3
NG_FAST = 7
SLOW_CORE = 1
TOTCH = EPAD // CH


def _agg_body(hs, src2, dst2, w2, out, acc,
              sm, dm, wm, rowsA, rowsB, gsA, gsB, ssA, ssB):
    c = lax.axis_index("c")
    s = lax.axis_index("s")
    nck_s = NG_SLOW * GRP
    nck_f = NG_FAST * GRP
    if SLOW_CORE == 0:
        cbase = jnp.where(c == 0, s * nck_s, NS * nck_s + s * nck_f)
        ngrp = jnp.where(c == 0, NG_SLOW, NG_FAST)
    else:
        cbase = jnp.where(c == 0, s * nck_f, NS * nck_f + s * nck_s)
        ngrp = jnp.where(c == 0, NG_FAST, NG_SLOW)

    z16 = jnp.zeros((16,), _f32)

    def zrow(i, carry):
        for f in range(D // 16):
            rowsA[i, pl.ds(f * 16, 16)] = z16
        return carry

    lax.fori_loop(0, CH, zrow, 0)
    for j in range(RPT // CH):
        pltpu.sync_copy(rowsA, acc.at[pl.ds(s * RPT + j * CH, CH)])
    plsc.subcore_barrier()

    def scale(rows, k):
        def group(g, c2):
            wvec = wm[k, pl.ds(g * 16, 16)]
            base = g * 16
            for e in range(16):
                sc = wvec[e]
                for f in range(D // 16):
                    rows[base + e, pl.ds(f * 16, 16)] = (
                        rows[base + e, pl.ds(f * 16, 16)] * sc
                    )
            return c2

        lax.fori_loop(0, CH // 16, group, 0)

    def grp(q, carry):
        row0 = cbase + q * GRP
        pltpu.sync_copy(src2.at[pl.ds(row0, GRP)], sm)
        pltpu.sync_copy(dst2.at[pl.ds(row0, GRP)], dm)
        pltpu.sync_copy(w2.at[pl.ds(row0, GRP)], wm)
        pltpu.async_copy(hs.at[sm.at[0]], rowsA, gsA)

        def pair(j, c1):
            k0 = 2 * j
            k1 = 2 * j + 1

            @pl.when(j > 0)
            def _():
                pltpu.make_async_copy(rowsB, acc.at[dm.at[k1]], ssB).wait()

            pltpu.make_async_copy(hs.at[sm.at[k0]], rowsA, gsA).wait()
            pltpu.async_copy(hs.at[sm.at[k1]], rowsB, gsB)
            scale(rowsA, k0)
            pltpu.async_copy(rowsA, acc.at[dm.at[k0]], ssA, add=True)
            pltpu.make_async_copy(hs.at[sm.at[k1]], rowsB, gsB).wait()
            pltpu.make_async_copy(rowsA, acc.at[dm.at[k0]], ssA).wait()

            @pl.when(j < NPAIR - 1)
            def _():
                pltpu.async_copy(hs.at[sm.at[k0 + 2]], rowsA, gsA)

            scale(rowsB, k1)
            pltpu.async_copy(rowsB, acc.at[dm.at[k1]], ssB, add=True)
            return c1

        lax.fori_loop(0, NPAIR, pair, 0)
        pltpu.make_async_copy(rowsB, acc.at[dm.at[0]], ssB).wait()
        return carry

    lax.fori_loop(0, ngrp, grp, 0)
    plsc.subcore_barrier()
    pltpu.sync_copy(acc.at[pl.ds(s * RPT, RPT)], out.at[c, pl.ds(s * RPT, RPT)])


_agg_call = functools.partial(
    pl.kernel,
    out_type=jax.ShapeDtypeStruct((NC, NPAD, D), _f32),
    mesh=_mesh,
    scratch_types=[
        pltpu.VMEM_SHARED((NPAD, D), _f32),
        pltpu.VMEM((GRP, CH), jnp.int32),
        pltpu.VMEM((GRP, CH), jnp.int32),
        pltpu.VMEM((GRP, CH), _f32),
        pltpu.VMEM((CH, D), _f32),
        pltpu.VMEM((CH, D), _f32),
        pltpu.SemaphoreType.DMA,
        pltpu.SemaphoreType.DMA,
        pltpu.SemaphoreType.DMA,
        pltpu.SemaphoreType.DMA,
    ],
)(_agg_body)


def _k0_body(x_ref, w_ref, degp_ref, hs_ref, dinv_ref):
    deg = jnp.sum(degp_ref[...], axis=0) + 1.0
    dinv = jnp.where(deg > 0, lax.rsqrt(deg), 0.0)
    h = jnp.dot(x_ref[...], w_ref[...], preferred_element_type=_f32)
    hs_ref[...] = h * dinv[:N, :]
    dinv_ref[...] = dinv


def _k0_call(x, w1, degp3):
    return pl.pallas_call(
        _k0_body,
        out_shape=(
            jax.ShapeDtypeStruct((N, D), _f32),
            jax.ShapeDtypeStruct((NPAD, 1), _f32),
        ),
    )(x, w1, degp3)


def _trans_body(parts_ref, hs_ref, dinv_ref, b_ref, g_ref, be_ref, w_ref, out_ref):
    dv = dinv_ref[:N, :]
    t = dv * (parts_ref[0, :N, :] + parts_ref[1, :N, :] + hs_ref[...]) + b_ref[...]
    r = jnp.maximum(t, 0.0)
    mu = jnp.mean(r, axis=0, keepdims=True)
    var = jnp.mean((r - mu) ** 2, axis=0, keepdims=True)
    y = (r - mu) * lax.rsqrt(var + 1e-5) * g_ref[...] + be_ref[...]
    h = jnp.dot(y, w_ref[...], preferred_element_type=_f32)
    out_ref[...] = h * dv


def _trans_call(parts, hs, dinv, b, g, be, w):
    return pl.pallas_call(
        _trans_body,
        out_shape=jax.ShapeDtypeStruct((N, D), _f32),
    )(parts, hs, dinv, b, g, be, w)


def _final_body(parts_ref, hs_ref, dinv_ref, b_ref, out_ref):
    dv = dinv_ref[:N, :]
    out_ref[...] = (
        dv * (parts_ref[0, :N, :] + parts_ref[1, :N, :] + hs_ref[...]) + b_ref[...]
    )


def _final_call(parts, hs, dinv, b):
    return pl.pallas_call(
        _final_body,
        out_shape=jax.ShapeDtypeStruct((N, D), _f32),
    )(parts, hs, dinv, b)


def kernel(x, edge_index, edge_weight, W1, b1, g1, be1, W2, b2, g2, be2, W3, b3):
    src = edge_index[0]
    dst = edge_index[1]
    pad = EPAD - E
    zi = jnp.zeros((pad,), jnp.int32)
    srcp = jnp.concatenate([src, zi])
    dstp = jnp.concatenate([dst, zi])
    wp = jnp.concatenate([edge_weight, jnp.zeros((pad,), _f32)])
    dst3 = dstp.reshape(NW, NCHUNK, CH)
    w3 = wp.reshape(NW, NCHUNK, CH)
    src2 = srcp.reshape(TOTCH, CH)
    dst2 = dstp.reshape(TOTCH, CH)
    w2 = wp.reshape(TOTCH, CH)
    b1r, g1r, be1r = b1.reshape(1, D), g1.reshape(1, D), be1.reshape(1, D)
    b2r, g2r, be2r = b2.reshape(1, D), g2.reshape(1, D), be2.reshape(1, D)
    b3r = b3.reshape(1, D)

    deg_parts = _deg_call(dst3, w3)
    degp3 = deg_parts.reshape(NC, NPAD, 1)
    h1s, dinv = _k0_call(x, W1, degp3)
    parts = _agg_call(h1s, src2, dst2, w2)
    h2s = _trans_call(parts, h1s, dinv, b1r, g1r, be1r, W2)
    parts = _agg_call(h2s, src2, dst2, w2)
    h3s = _trans_call(parts, h2s, dinv, b2r, g2r, be2r, W3)
    parts = _agg_call(h3s, src2, dst2, w2)
    return _final_call(parts, h3s, dinv, b3r)
